# baseline (speedup 1.0000x reference)
import functools

import jax
import jax.numpy as jnp
from jax import lax
from jax.experimental import pallas as pl
from jax.experimental.pallas import tpu as pltpu

_N = 50000


def _head_body(xcol_ref, y_any, w1_ref, b1_ref, w2_ref, b2_ref, o_ref,
               yrow, sem):
    blk = xcol_ref[...]
    m, c = blk.shape
    idx = (lax.broadcasted_iota(jnp.int32, (m, c), 0) * 128
           + lax.broadcasted_iota(jnp.int32, (m, c), 1))
    agent = jnp.max(jnp.where(blk == 1.0, idx, -1))
    row = jnp.where(agent < 0, _N - 1, agent)
    pltpu.make_async_copy(y_any.at[pl.ds(row, 1)], yrow, sem).start()
    pltpu.make_async_copy(y_any.at[pl.ds(row, 1)], yrow, sem).wait()
    h = jax.nn.relu(yrow[...] @ w1_ref[...] + b1_ref[...])
    o_ref[...] = h @ w2_ref[...] + b2_ref[...]


def _head(x3, y, fc1_w, fc1_b, fc2_w, fc2_b):
    pad = (-_N) % 128
    xcol = jnp.concatenate([x3, jnp.zeros((pad,), jnp.float32)]).reshape(-1, 128)
    out = pl.pallas_call(
        _head_body,
        out_shape=jax.ShapeDtypeStruct((1, 1), jnp.float32),
        in_specs=[
            pl.BlockSpec(memory_space=pltpu.MemorySpace.VMEM),
            pl.BlockSpec(memory_space=pltpu.MemorySpace.HBM),
            pl.BlockSpec(memory_space=pltpu.MemorySpace.VMEM),
            pl.BlockSpec(memory_space=pltpu.MemorySpace.VMEM),
            pl.BlockSpec(memory_space=pltpu.MemorySpace.VMEM),
            pl.BlockSpec(memory_space=pltpu.MemorySpace.VMEM),
        ],
        out_specs=pl.BlockSpec(memory_space=pltpu.MemorySpace.VMEM),
        scratch_shapes=[
            pltpu.VMEM((1, 64), jnp.float32),
            pltpu.SemaphoreType.DMA,
        ],
    )(xcol, y, fc1_w, fc1_b[None, :], fc2_w, fc2_b[None, :])
    return out.reshape(1)


def kernel(x, edges, params):
    num_nodes = x.shape[0]
    src = edges[0]
    dst = edges[1]
    deg = jnp.zeros((num_nodes,), jnp.float32).at[src].add(1.0)
    dinv = jnp.where(deg > 0, jax.lax.rsqrt(jnp.where(deg > 0, deg, 1.0)), 0.0)
    norm = -(dinv[src] * dinv[dst])

    order = jnp.argsort(dst)
    src_s = src[order]
    dst_s = dst[order]
    norm_s = norm[order]

    y = x
    for l in range(8):
        W = params['W%d' % l]
        b = params['b%d' % l]
        msg = norm_s[:, None] * y[src_s]
        tx1 = jnp.zeros((num_nodes, y.shape[1]), jnp.float32).at[dst_s].add(msg)
        y = jax.nn.relu(y @ W[0] + tx1 @ W[1] + b)

    return _head(x[:, 3], y, params['fc1_w'], params['fc1_b'],
                 params['fc2_w'], params['fc2_b'])


# speedup vs baseline: 1.5584x; 1.5584x over previous
import jax
import jax.numpy as jnp
from jax import lax
from jax.experimental import pallas as pl
from jax.experimental.pallas import tpu as pltpu

_N = 50000
_E = 800000
_NPAD = 50176


def _dinv_body(d_ref, o_ref):
    d = d_ref[...]
    pos = d > 0
    o_ref[...] = jnp.where(pos, lax.rsqrt(jnp.where(pos, d, 1.0)), 0.0)


def _dinv(deg):
    out = pl.pallas_call(
        _dinv_body,
        out_shape=jax.ShapeDtypeStruct((_NPAD // 512, 512), jnp.float32),
    )(deg.reshape(_NPAD // 512, 512))
    return out.reshape(_NPAD)


def _scale_body(x_ref, d_ref, o_ref):
    o_ref[...] = x_ref[...] * d_ref[...]


def _scale(xp, dinv_b):
    return pl.pallas_call(
        _scale_body,
        grid=(_NPAD // 512,),
        in_specs=[
            pl.BlockSpec((512, 64), lambda i: (i, 0)),
            pl.BlockSpec((512, 64), lambda i: (i, 0)),
        ],
        out_specs=pl.BlockSpec((512, 64), lambda i: (i, 0)),
        out_shape=jax.ShapeDtypeStruct((_NPAD, 64), jnp.float32),
    )(xp, dinv_b)


def _dense_body(h_ref, s_ref, db_ref, w0_ref, w1_ref, b_ref, o_ref, hs_ref):
    db = db_ref[...]
    t = db * s_ref[...]
    a = jnp.dot(h_ref[...], w0_ref[...], preferred_element_type=jnp.float32)
    c2 = jnp.dot(t, w1_ref[...], preferred_element_type=jnp.float32)
    out = jnp.maximum(a - c2 + b_ref[...], 0.0)
    o_ref[...] = out
    hs_ref[...] = out * db


def _dense(h, seg, dinv_b, w0, w1, b):
    return pl.pallas_call(
        _dense_body,
        grid=(_NPAD // 512,),
        in_specs=[
            pl.BlockSpec((512, 64), lambda i: (i, 0)),
            pl.BlockSpec((512, 64), lambda i: (i, 0)),
            pl.BlockSpec((512, 64), lambda i: (i, 0)),
            pl.BlockSpec((64, 64), lambda i: (0, 0)),
            pl.BlockSpec((64, 64), lambda i: (0, 0)),
            pl.BlockSpec((1, 64), lambda i: (0, 0)),
        ],
        out_specs=[
            pl.BlockSpec((512, 64), lambda i: (i, 0)),
            pl.BlockSpec((512, 64), lambda i: (i, 0)),
        ],
        out_shape=[
            jax.ShapeDtypeStruct((_NPAD, 64), jnp.float32),
            jax.ShapeDtypeStruct((_NPAD, 64), jnp.float32),
        ],
    )(h, seg, dinv_b, w0, w1, b)


def _head_body(xcol_ref, y_any, w1_ref, b1_ref, w2_ref, b2_ref, o_ref,
               yrow, sem):
    blk = xcol_ref[...]
    mrows, cc = blk.shape
    idx = (lax.broadcasted_iota(jnp.int32, (mrows, cc), 0) * 128
           + lax.broadcasted_iota(jnp.int32, (mrows, cc), 1))
    agent = jnp.max(jnp.where(blk == 1.0, idx, -1))
    row = jnp.where(agent < 0, _N - 1, agent)
    pltpu.make_async_copy(y_any.at[pl.ds(row, 1)], yrow, sem).start()
    pltpu.make_async_copy(y_any.at[pl.ds(row, 1)], yrow, sem).wait()
    h = jax.nn.relu(yrow[...] @ w1_ref[...] + b1_ref[...])
    o_ref[...] = h @ w2_ref[...] + b2_ref[...]


def _head(x3, y, fc1_w, fc1_b, fc2_w, fc2_b):
    pad = (-_N) % 128
    xcol = jnp.concatenate([x3, jnp.zeros((pad,), jnp.float32)]).reshape(-1, 128)
    out = pl.pallas_call(
        _head_body,
        out_shape=jax.ShapeDtypeStruct((1, 1), jnp.float32),
        in_specs=[
            pl.BlockSpec(memory_space=pltpu.MemorySpace.VMEM),
            pl.BlockSpec(memory_space=pltpu.MemorySpace.HBM),
            pl.BlockSpec(memory_space=pltpu.MemorySpace.VMEM),
            pl.BlockSpec(memory_space=pltpu.MemorySpace.VMEM),
            pl.BlockSpec(memory_space=pltpu.MemorySpace.VMEM),
            pl.BlockSpec(memory_space=pltpu.MemorySpace.VMEM),
        ],
        out_specs=pl.BlockSpec(memory_space=pltpu.MemorySpace.VMEM),
        scratch_shapes=[
            pltpu.VMEM((1, 64), jnp.float32),
            pltpu.SemaphoreType.DMA,
        ],
    )(xcol, y, fc1_w, fc1_b[None, :], fc2_w, fc2_b[None, :])
    return out.reshape(1)


def kernel(x, edges, params):
    src = edges[0].astype(jnp.int32)
    dst = edges[1].astype(jnp.int32)
    dst_s, src_s = lax.sort((dst, src), num_keys=1)

    deg = jnp.zeros((_NPAD,), jnp.float32).at[src].add(1.0)
    dinv = _dinv(deg)
    dinv_b = jnp.broadcast_to(dinv[:, None], (_NPAD, 64))

    xp = jnp.zeros((_NPAD, 64), jnp.float32).at[:_N, :8].set(x)
    hs = _scale(xp, dinv_b)
    h = xp
    for l in range(8):
        W = params['W%d' % l]
        if l == 0:
            w0 = jnp.zeros((64, 64), jnp.float32).at[:8].set(W[0])
            w1 = jnp.zeros((64, 64), jnp.float32).at[:8].set(W[1])
        else:
            w0, w1 = W[0], W[1]
        seg = jnp.zeros((_NPAD, 64), jnp.float32).at[dst_s].add(hs[src_s])
        h, hs = _dense(h, seg, dinv_b, w0, w1, params['b%d' % l][None, :])

    return _head(x[:, 3], h, params['fc1_w'], params['fc1_b'],
                 params['fc2_w'], params['fc2_b'])


# speedup vs baseline: 1.6173x; 1.0378x over previous
import jax
import jax.numpy as jnp
from jax import lax
from jax.experimental import pallas as pl
from jax.experimental.pallas import tpu as pltpu

_N = 50000
_E = 800000
_NPAD = 50176


def _dinv_body(d_ref, o_ref):
    d = d_ref[...]
    pos = d > 0
    o_ref[...] = jnp.where(pos, lax.rsqrt(jnp.where(pos, d, 1.0)), 0.0)


def _dinv(deg):
    out = pl.pallas_call(
        _dinv_body,
        out_shape=jax.ShapeDtypeStruct((_NPAD // 512, 512), jnp.float32),
    )(deg.reshape(_NPAD // 512, 512))
    return out.reshape(_NPAD)


def _scale_body(x_ref, d_ref, o_ref):
    o_ref[...] = x_ref[...] * d_ref[...]


def _scale(xp, dinv_b):
    return pl.pallas_call(
        _scale_body,
        grid=(_NPAD // 512,),
        in_specs=[
            pl.BlockSpec((512, 64), lambda i: (i, 0)),
            pl.BlockSpec((512, 64), lambda i: (i, 0)),
        ],
        out_specs=pl.BlockSpec((512, 64), lambda i: (i, 0)),
        out_shape=jax.ShapeDtypeStruct((_NPAD, 64), jnp.float32),
    )(xp, dinv_b)


def _dense_body(h_ref, s_ref, db_ref, w0_ref, w1_ref, b_ref, o_ref, hs_ref):
    db = db_ref[...]
    t = db * s_ref[...]
    a = jnp.dot(h_ref[...], w0_ref[...], preferred_element_type=jnp.float32)
    c2 = jnp.dot(t, w1_ref[...], preferred_element_type=jnp.float32)
    out = jnp.maximum(a - c2 + b_ref[...], 0.0)
    o_ref[...] = out
    hs_ref[...] = out * db


def _dense(h, seg, dinv_b, w0, w1, b):
    return pl.pallas_call(
        _dense_body,
        grid=(_NPAD // 512,),
        in_specs=[
            pl.BlockSpec((512, 64), lambda i: (i, 0)),
            pl.BlockSpec((512, 64), lambda i: (i, 0)),
            pl.BlockSpec((512, 64), lambda i: (i, 0)),
            pl.BlockSpec((64, 64), lambda i: (0, 0)),
            pl.BlockSpec((64, 64), lambda i: (0, 0)),
            pl.BlockSpec((1, 64), lambda i: (0, 0)),
        ],
        out_specs=[
            pl.BlockSpec((512, 64), lambda i: (i, 0)),
            pl.BlockSpec((512, 64), lambda i: (i, 0)),
        ],
        out_shape=[
            jax.ShapeDtypeStruct((_NPAD, 64), jnp.float32),
            jax.ShapeDtypeStruct((_NPAD, 64), jnp.float32),
        ],
    )(h, seg, dinv_b, w0, w1, b)


def _head_body(xcol_ref, y_any, w1_ref, b1_ref, w2_ref, b2_ref, o_ref,
               yrow, sem):
    blk = xcol_ref[...]
    mrows, cc = blk.shape
    idx = (lax.broadcasted_iota(jnp.int32, (mrows, cc), 0) * 128
           + lax.broadcasted_iota(jnp.int32, (mrows, cc), 1))
    agent = jnp.max(jnp.where(blk == 1.0, idx, -1))
    row = jnp.where(agent < 0, _N - 1, agent)
    pltpu.make_async_copy(y_any.at[pl.ds(row, 1)], yrow, sem).start()
    pltpu.make_async_copy(y_any.at[pl.ds(row, 1)], yrow, sem).wait()
    h = jax.nn.relu(yrow[...] @ w1_ref[...] + b1_ref[...])
    o_ref[...] = h @ w2_ref[...] + b2_ref[...]


def _head(x3, y, fc1_w, fc1_b, fc2_w, fc2_b):
    pad = (-_N) % 128
    xcol = jnp.concatenate([x3, jnp.zeros((pad,), jnp.float32)]).reshape(-1, 128)
    out = pl.pallas_call(
        _head_body,
        out_shape=jax.ShapeDtypeStruct((1, 1), jnp.float32),
        in_specs=[
            pl.BlockSpec(memory_space=pltpu.MemorySpace.VMEM),
            pl.BlockSpec(memory_space=pltpu.MemorySpace.HBM),
            pl.BlockSpec(memory_space=pltpu.MemorySpace.VMEM),
            pl.BlockSpec(memory_space=pltpu.MemorySpace.VMEM),
            pl.BlockSpec(memory_space=pltpu.MemorySpace.VMEM),
            pl.BlockSpec(memory_space=pltpu.MemorySpace.VMEM),
        ],
        out_specs=pl.BlockSpec(memory_space=pltpu.MemorySpace.VMEM),
        scratch_shapes=[
            pltpu.VMEM((1, 64), jnp.float32),
            pltpu.SemaphoreType.DMA,
        ],
    )(xcol, y, fc1_w, fc1_b[None, :], fc2_w, fc2_b[None, :])
    return out.reshape(1)


def kernel(x, edges, params):
    src_s = edges[0].astype(jnp.int32)
    dst_s = edges[1].astype(jnp.int32)

    deg = jnp.zeros((_NPAD,), jnp.float32).at[src_s].add(1.0)
    dinv = _dinv(deg)
    dinv_b = jnp.broadcast_to(dinv[:, None], (_NPAD, 64))

    xp = jnp.zeros((_NPAD, 64), jnp.float32).at[:_N, :8].set(x)
    hs = _scale(xp, dinv_b)
    h = xp
    for l in range(8):
        W = params['W%d' % l]
        if l == 0:
            w0 = jnp.zeros((64, 64), jnp.float32).at[:8].set(W[0])
            w1 = jnp.zeros((64, 64), jnp.float32).at[:8].set(W[1])
        else:
            w0, w1 = W[0], W[1]
        seg = jnp.zeros((_NPAD, 64), jnp.float32).at[dst_s].add(hs[src_s])
        h, hs = _dense(h, seg, dinv_b, w0, w1, params['b%d' % l][None, :])

    return _head(x[:, 3], h, params['fc1_w'], params['fc1_b'],
                 params['fc2_w'], params['fc2_b'])
